# baseline (speedup 1.0000x reference)
import functools
import math

import jax
import jax.numpy as jnp
from jax import lax
from jax.experimental import pallas as pl
from jax.experimental.pallas import tpu as pltpu
from jax.experimental.pallas import tpu_sc as plsc

N_NODES = 10000
N_EDGES = 320000
D_IN = 128
D_HID = 128
D_OUT = 16

NC = 2
NS = 16
NW = NC * NS
NP = 10240
CHUNK = 120
CPT = 168
EP = NS * CPT * CHUNK
TOTAL_CH = EP // CHUNK
EPW0 = EP // NW
ROWS_PER_TILE = NP // NS
PART_BUF = 32

A0_128, A1_128 = 128, 40
A0_16, A1_16 = 112, 56

_mesh = plsc.VectorSubcoreMesh(core_axis_name="c", subcore_axis_name="s")
_sc_params = pltpu.CompilerParams(needs_layout_passes=False)


def _parts(n):
    k = math.ceil(n / PART_BUF)
    base = (n // k) & ~7
    sizes = [base] * k
    leftover = n - base * k
    i = 0
    while leftover > 0:
        sizes[i % k] += 8
        leftover -= 8
        i += 1
    assert sum(sizes) == n and all(p % 8 == 0 and p <= PART_BUF for p in sizes)
    return sizes


@functools.partial(
    pl.kernel,
    out_type=jax.ShapeDtypeStruct((NW, NP), jnp.float32),
    mesh=_mesh,
    scratch_types=[
        pltpu.VMEM((EPW0,), jnp.int32),
        pltpu.VMEM((NP,), jnp.float32),
        pltpu.SemaphoreType.DMA,
    ],
    compiler_params=_sc_params,
)
def _deg_kernel(dst_hbm, out_hbm, dst_v, deg_v, sem):
    c = lax.axis_index("c")
    s = lax.axis_index("s")
    wid = s * NC + c

    cp = pltpu.async_copy(dst_hbm.at[wid], dst_v, sem)

    zeros16 = jnp.zeros((16,), jnp.float32)

    def zero_body(i, _):
        deg_v[pl.ds(i * 16, 16)] = zeros16
        return None

    lax.fori_loop(0, NP // 16, zero_body, None)
    cp.wait()

    ones16 = jnp.ones((16,), jnp.float32)

    def add_body(i, _):
        idx = dst_v[pl.ds(i * 16, 16)]
        plsc.addupdate_scatter(deg_v, [idx], ones16)
        return None

    lax.fori_loop(0, EPW0 // 16, add_body, None)

    pltpu.sync_copy(deg_v, out_hbm.at[wid])


def _make_agg_kernel(D, a0, a1):
    params = (
        _sc_params
        if D % 128 == 0
        else pltpu.CompilerParams(
            needs_layout_passes=False, use_tc_tiling_on_sc=False
        )
    )
    assert NS * (a0 + a1) == TOTAL_CH

    @functools.partial(
        pl.kernel,
        out_type=jax.ShapeDtypeStruct((NC, NP, D), jnp.float32),
        mesh=_mesh,
        scratch_types=[
            pltpu.VMEM((2, PART_BUF, CHUNK), jnp.int32),
            pltpu.VMEM((2, PART_BUF, CHUNK), jnp.int32),
            pltpu.VMEM((CHUNK, D), jnp.float32),
            pltpu.VMEM((CHUNK, D), jnp.float32),
            pltpu.VMEM_SHARED((NP, D), jnp.float32),
            pltpu.SemaphoreType.DMA,
            pltpu.SemaphoreType.DMA,
            pltpu.SemaphoreType.DMA,
            pltpu.SemaphoreType.DMA,
        ],
        compiler_params=params,
    )
    def agg(hp_hbm, src_hbm, dst_hbm, out_hbm, src_v, dst_v, buf_a, buf_b,
            acc_s, sem_a, sem_b, sem_i0, sem_i1):
        c = lax.axis_index("c")
        s = lax.axis_index("s")
        isem = (sem_i0, sem_i1)

        def idx_copies(slot, psz, chunk_start):
            return (
                pltpu.make_async_copy(
                    src_hbm.at[pl.ds(chunk_start, psz)],
                    src_v.at[slot, pl.ds(0, psz)],
                    isem[slot],
                ),
                pltpu.make_async_copy(
                    dst_hbm.at[pl.ds(chunk_start, psz)],
                    dst_v.at[slot, pl.ds(0, psz)],
                    isem[slot],
                ),
            )

        def fire_idx(slot, psz, chunk_start):
            for cp in idx_copies(slot, psz, chunk_start):
                cp.start()

        def wait_idx(slot, psz, chunk_start):
            for cp in idx_copies(slot, psz, chunk_start):
                cp.wait()

        parts0 = _parts(a0)
        parts1 = _parts(a1)
        base0 = s * a0
        base1 = NS * a0 + s * a1

        @pl.when(c == 0)
        def _():
            fire_idx(0, parts0[0], base0)

        @pl.when(c == 1)
        def _():
            fire_idx(0, parts1[0], base1)

        zeros16 = jnp.zeros((16,), jnp.float32)
        ncol = D // 16

        def zero_body(i, _):
            r = i // ncol
            k = i % ncol
            buf_a[r, pl.ds(k * 16, 16)] = zeros16
            return None

        lax.fori_loop(0, CHUNK * ncol, zero_body, None)

        zsizes = []
        left = ROWS_PER_TILE
        while left > 0:
            zsizes.append(min(CHUNK, left))
            left -= zsizes[-1]
        off = 0
        for zs in zsizes:
            pltpu.sync_copy(
                buf_a.at[pl.ds(0, zs)],
                acc_s.at[pl.ds(s * ROWS_PER_TILE + off, zs)],
            )
            off += zs
        plsc.subcore_barrier()

        def gather(slot, j, buf, sem):
            pltpu.async_copy(hp_hbm.at[src_v.at[slot, j]], buf, sem)

        def gather_wait(slot, j, buf, sem):
            pltpu.make_async_copy(
                hp_hbm.at[src_v.at[slot, j]], buf, sem
            ).wait()

        def scatter(slot, j, buf):
            pltpu.sync_copy(buf, acc_s.at[dst_v.at[slot, j]], add=True)

        def main_loop(parts, base):
            off = 0
            for pi, psz in enumerate(parts):
                slot = pi % 2
                wait_idx(slot, psz, base + off)
                if pi + 1 < len(parts):
                    fire_idx(1 - slot, parts[pi + 1], base + off + psz)

                gather(slot, 0, buf_a, sem_a)

                def pair_body(i, _):
                    gather(slot, 2 * i + 1, buf_b, sem_b)
                    gather_wait(slot, 2 * i, buf_a, sem_a)
                    scatter(slot, 2 * i, buf_a)

                    @pl.when(i < psz // 2 - 1)
                    def _():
                        gather(slot, 2 * i + 2, buf_a, sem_a)

                    gather_wait(slot, 2 * i + 1, buf_b, sem_b)
                    scatter(slot, 2 * i + 1, buf_b)
                    return None

                lax.fori_loop(0, psz // 2, pair_body, None)
                off += psz

        @pl.when(c == 0)
        def _():
            main_loop(parts0, base0)

        @pl.when(c == 1)
        def _():
            main_loop(parts1, base1)

        plsc.subcore_barrier()

        off = 0
        for zs in zsizes:
            base = s * ROWS_PER_TILE + off
            pltpu.sync_copy(acc_s.at[pl.ds(base, zs)], buf_a.at[pl.ds(0, zs)])
            pltpu.sync_copy(buf_a.at[pl.ds(0, zs)], out_hbm.at[c, pl.ds(base, zs)])
            off += zs

    return agg


_agg128 = _make_agg_kernel(D_HID, A0_128, A1_128)
_agg16 = _make_agg_kernel(D_OUT, A0_16, A1_16)


_BR = 1024
_NGRID = NP // _BR


def _dinv_from_parts(degp):
    deg = jnp.sum(degp, axis=0) + 1.0
    return lax.rsqrt(jnp.maximum(deg, 1.0))


def _t1_body(x_ref, w_ref, degp_ref, out_ref):
    dinv = _dinv_from_parts(degp_ref[...])
    h = jnp.dot(x_ref[...], w_ref[...], preferred_element_type=jnp.float32)
    out_ref[...] = h * dinv[:, None]


def _t1(x, W1, degp):
    return pl.pallas_call(
        _t1_body,
        out_shape=jax.ShapeDtypeStruct((NP, D_HID), jnp.float32),
        grid=(_NGRID,),
        in_specs=[
            pl.BlockSpec((_BR, D_IN), lambda r: (r, 0)),
            pl.BlockSpec((D_IN, D_HID), lambda r: (0, 0)),
            pl.BlockSpec((NW, _BR), lambda r: (0, r)),
        ],
        out_specs=pl.BlockSpec((_BR, D_HID), lambda r: (r, 0)),
    )(x, W1, degp)


def _t2_body(aggp_ref, hp_ref, degp_ref, b1_ref, w2_ref, out_ref):
    dinv = _dinv_from_parts(degp_ref[...])
    tot = aggp_ref[0] + aggp_ref[1] + hp_ref[...]
    a = tot * dinv[:, None] + b1_ref[...][None, :]
    h1 = jnp.maximum(a, 0.0)
    h2 = jnp.dot(h1, w2_ref[...], preferred_element_type=jnp.float32)
    out_ref[...] = h2 * dinv[:, None]


def _t2(aggp, hp, degp, b1, W2):
    return pl.pallas_call(
        _t2_body,
        out_shape=jax.ShapeDtypeStruct((NP, D_OUT), jnp.float32),
        grid=(_NGRID,),
        in_specs=[
            pl.BlockSpec((NC, _BR, D_HID), lambda r: (0, r, 0)),
            pl.BlockSpec((_BR, D_HID), lambda r: (r, 0)),
            pl.BlockSpec((NW, _BR), lambda r: (0, r)),
            pl.BlockSpec((D_HID,), lambda r: (0,)),
            pl.BlockSpec((D_HID, D_OUT), lambda r: (0, 0)),
        ],
        out_specs=pl.BlockSpec((_BR, D_OUT), lambda r: (r, 0)),
    )(aggp, hp, degp, b1, W2)


def _t3_body(aggp_ref, hp_ref, degp_ref, b2_ref, out_ref):
    dinv = _dinv_from_parts(degp_ref[...])
    tot = aggp_ref[0] + aggp_ref[1] + hp_ref[...]
    o = tot * dinv[:, None] + b2_ref[...][None, :]
    m = jnp.max(o, axis=1, keepdims=True)
    lse = m + jnp.log(jnp.sum(jnp.exp(o - m), axis=1, keepdims=True))
    out_ref[...] = o - lse


def _t3(aggp, hp, degp, b2):
    return pl.pallas_call(
        _t3_body,
        out_shape=jax.ShapeDtypeStruct((N_NODES, D_OUT), jnp.float32),
        grid=(_NGRID,),
        in_specs=[
            pl.BlockSpec((NC, _BR, D_OUT), lambda r: (0, r, 0)),
            pl.BlockSpec((_BR, D_OUT), lambda r: (r, 0)),
            pl.BlockSpec((NW, _BR), lambda r: (0, r)),
            pl.BlockSpec((D_OUT,), lambda r: (0,)),
        ],
        out_specs=pl.BlockSpec((_BR, D_OUT), lambda r: (r, 0)),
    )(aggp, hp, degp, b2)


def kernel(x, edge_index, W1, b1, W2, b2):
    pad_e = EP - N_EDGES
    src_c = jnp.concatenate(
        [edge_index[0], jnp.full((pad_e,), N_NODES, jnp.int32)]
    ).reshape(TOTAL_CH, CHUNK)
    dst_flat = jnp.concatenate(
        [edge_index[1], jnp.full((pad_e,), N_NODES, jnp.int32)]
    )
    dst_c = dst_flat.reshape(TOTAL_CH, CHUNK)

    degp = _deg_kernel(dst_flat.reshape(NW, EPW0))
    h1p = _t1(x, W1, degp)
    agg1 = _agg128(h1p, src_c, dst_c)
    h2p = _t2(agg1, h1p, degp, b1, W2)
    agg2 = _agg16(h2p, src_c, dst_c)
    return _t3(agg2, h2p, degp, b2)


# speedup vs baseline: 1.0135x; 1.0135x over previous
import functools
import math

import jax
import jax.numpy as jnp
from jax import lax
from jax.experimental import pallas as pl
from jax.experimental.pallas import tpu as pltpu
from jax.experimental.pallas import tpu_sc as plsc

N_NODES = 10000
N_EDGES = 320000
D_IN = 128
D_HID = 128
D_OUT = 16

NC = 2
NS = 16
NW = NC * NS
NP = 10240
CHUNK = 120
CPT = 168
EP = NS * CPT * CHUNK
TOTAL_CH = EP // CHUNK
EPW0 = EP // NW
ROWS_PER_TILE = NP // NS
PART_BUF = 32
NBUF = 2

A0_128, A1_128 = 128, 40
A0_16, A1_16 = 104, 64

_mesh = plsc.VectorSubcoreMesh(core_axis_name="c", subcore_axis_name="s")
_sc_params = pltpu.CompilerParams(needs_layout_passes=False)


def _parts(n):
    k = math.ceil(n / PART_BUF)
    base = (n // k) & ~7
    sizes = [base] * k
    leftover = n - base * k
    i = 0
    while leftover > 0:
        sizes[i % k] += 8
        leftover -= 8
        i += 1
    assert sum(sizes) == n and all(p % 8 == 0 and p <= PART_BUF for p in sizes)
    return sizes


@functools.partial(
    pl.kernel,
    out_type=jax.ShapeDtypeStruct((NW, NP), jnp.float32),
    mesh=_mesh,
    scratch_types=[
        pltpu.VMEM((EPW0,), jnp.int32),
        pltpu.VMEM((NP,), jnp.float32),
        pltpu.SemaphoreType.DMA,
    ],
    compiler_params=_sc_params,
)
def _deg_kernel(dst_hbm, out_hbm, dst_v, deg_v, sem):
    c = lax.axis_index("c")
    s = lax.axis_index("s")
    wid = s * NC + c

    cp = pltpu.async_copy(dst_hbm.at[wid], dst_v, sem)

    zeros16 = jnp.zeros((16,), jnp.float32)

    def zero_body(i, _):
        deg_v[pl.ds(i * 16, 16)] = zeros16
        return None

    lax.fori_loop(0, NP // 16, zero_body, None)
    cp.wait()

    ones16 = jnp.ones((16,), jnp.float32)

    def add_body(i, _):
        idx = dst_v[pl.ds(i * 16, 16)]
        plsc.addupdate_scatter(deg_v, [idx], ones16)
        return None

    lax.fori_loop(0, EPW0 // 16, add_body, None)

    pltpu.sync_copy(deg_v, out_hbm.at[wid])


def _make_agg_kernel(D, a0, a1):
    params = (
        _sc_params
        if D % 128 == 0
        else pltpu.CompilerParams(
            needs_layout_passes=False, use_tc_tiling_on_sc=False
        )
    )
    assert NS * (a0 + a1) == TOTAL_CH

    @functools.partial(
        pl.kernel,
        out_type=jax.ShapeDtypeStruct((NC, NP, D), jnp.float32),
        mesh=_mesh,
        scratch_types=[
            pltpu.VMEM((2, PART_BUF, CHUNK), jnp.int32),
            pltpu.VMEM((2, PART_BUF, CHUNK), jnp.int32),
            pltpu.VMEM((CHUNK, D), jnp.float32),
            pltpu.VMEM((CHUNK, D), jnp.float32),
            pltpu.VMEM_SHARED((NP, D), jnp.float32),
            pltpu.SemaphoreType.DMA,
            pltpu.SemaphoreType.DMA,
            pltpu.SemaphoreType.DMA,
            pltpu.SemaphoreType.DMA,
            pltpu.SemaphoreType.DMA,
            pltpu.SemaphoreType.DMA,
        ],
        compiler_params=params,
    )
    def agg(hp_hbm, src_hbm, dst_hbm, out_hbm, src_v, dst_v, buf0, buf1,
            acc_s, gsem0, gsem1, ssem0, ssem1, sem_i0, sem_i1):
        bufs = (buf0, buf1)
        gsems = (gsem0, gsem1)
        ssems = (ssem0, ssem1)
        c = lax.axis_index("c")
        s = lax.axis_index("s")
        isem = (sem_i0, sem_i1)

        def idx_copies(slot, psz, chunk_start):
            return (
                pltpu.make_async_copy(
                    src_hbm.at[pl.ds(chunk_start, psz)],
                    src_v.at[slot, pl.ds(0, psz)],
                    isem[slot],
                ),
                pltpu.make_async_copy(
                    dst_hbm.at[pl.ds(chunk_start, psz)],
                    dst_v.at[slot, pl.ds(0, psz)],
                    isem[slot],
                ),
            )

        def fire_idx(slot, psz, chunk_start):
            for cp in idx_copies(slot, psz, chunk_start):
                cp.start()

        def wait_idx(slot, psz, chunk_start):
            for cp in idx_copies(slot, psz, chunk_start):
                cp.wait()

        parts0 = _parts(a0)
        parts1 = _parts(a1)
        base0 = s * a0
        base1 = NS * a0 + s * a1

        @pl.when(c == 0)
        def _():
            fire_idx(0, parts0[0], base0)

        @pl.when(c == 1)
        def _():
            fire_idx(0, parts1[0], base1)

        zeros16 = jnp.zeros((16,), jnp.float32)
        ncol = D // 16

        def zero_body(i, _):
            r = i // ncol
            k = i % ncol
            bufs[0][r, pl.ds(k * 16, 16)] = zeros16
            return None

        lax.fori_loop(0, CHUNK * ncol, zero_body, None)

        zsizes = []
        left = ROWS_PER_TILE
        while left > 0:
            zsizes.append(min(CHUNK, left))
            left -= zsizes[-1]
        off = 0
        for zs in zsizes:
            pltpu.sync_copy(
                bufs[0].at[pl.ds(0, zs)],
                acc_s.at[pl.ds(s * ROWS_PER_TILE + off, zs)],
            )
            off += zs
        plsc.subcore_barrier()

        def gather_cp(slot, j, b):
            return pltpu.make_async_copy(
                hp_hbm.at[src_v.at[slot, j]], bufs[b], gsems[b]
            )

        def scatter_cp(slot, j, b):
            return pltpu.make_async_copy(
                bufs[b], acc_s.at[dst_v.at[slot, j]], ssems[b]
            )

        def main_loop(parts, base):
            off = 0
            for pi, psz in enumerate(parts):
                slot = pi % 2
                wait_idx(slot, psz, base + off)
                if pi + 1 < len(parts):
                    fire_idx(1 - slot, parts[pi + 1], base + off + psz)

                for b in range(NBUF):
                    gather_cp(slot, b, b).start()

                def ring_body(i, _):
                    for b in range(NBUF):
                        j = NBUF * i + b
                        gather_cp(slot, j, b).wait()
                        cp = scatter_cp(slot, j, b)
                        cp.start(add=True)
                        cp.wait()

                        @pl.when(j + NBUF < psz)
                        def _():
                            gather_cp(slot, j + NBUF, b).start()

                    return None

                lax.fori_loop(0, psz // NBUF, ring_body, None)
                off += psz

        @pl.when(c == 0)
        def _():
            main_loop(parts0, base0)

        @pl.when(c == 1)
        def _():
            main_loop(parts1, base1)

        plsc.subcore_barrier()

        off = 0
        for zs in zsizes:
            base = s * ROWS_PER_TILE + off
            pltpu.sync_copy(acc_s.at[pl.ds(base, zs)], bufs[0].at[pl.ds(0, zs)])
            pltpu.sync_copy(bufs[0].at[pl.ds(0, zs)], out_hbm.at[c, pl.ds(base, zs)])
            off += zs

    return agg


_agg128 = _make_agg_kernel(D_HID, A0_128, A1_128)
_agg16 = _make_agg_kernel(D_OUT, A0_16, A1_16)


_BR = 1024
_NGRID = NP // _BR


def _dinv_from_parts(degp):
    deg = jnp.sum(degp, axis=0) + 1.0
    return lax.rsqrt(jnp.maximum(deg, 1.0))


def _t1_body(x_ref, w_ref, degp_ref, out_ref):
    dinv = _dinv_from_parts(degp_ref[...])
    h = jnp.dot(x_ref[...], w_ref[...], preferred_element_type=jnp.float32)
    out_ref[...] = h * dinv[:, None]


def _t1(x, W1, degp):
    return pl.pallas_call(
        _t1_body,
        out_shape=jax.ShapeDtypeStruct((NP, D_HID), jnp.float32),
        grid=(_NGRID,),
        in_specs=[
            pl.BlockSpec((_BR, D_IN), lambda r: (r, 0)),
            pl.BlockSpec((D_IN, D_HID), lambda r: (0, 0)),
            pl.BlockSpec((NW, _BR), lambda r: (0, r)),
        ],
        out_specs=pl.BlockSpec((_BR, D_HID), lambda r: (r, 0)),
    )(x, W1, degp)


def _t2_body(aggp_ref, hp_ref, degp_ref, b1_ref, w2_ref, out_ref):
    dinv = _dinv_from_parts(degp_ref[...])
    tot = aggp_ref[0] + aggp_ref[1] + hp_ref[...]
    a = tot * dinv[:, None] + b1_ref[...][None, :]
    h1 = jnp.maximum(a, 0.0)
    h2 = jnp.dot(h1, w2_ref[...], preferred_element_type=jnp.float32)
    out_ref[...] = h2 * dinv[:, None]


def _t2(aggp, hp, degp, b1, W2):
    return pl.pallas_call(
        _t2_body,
        out_shape=jax.ShapeDtypeStruct((NP, D_OUT), jnp.float32),
        grid=(_NGRID,),
        in_specs=[
            pl.BlockSpec((NC, _BR, D_HID), lambda r: (0, r, 0)),
            pl.BlockSpec((_BR, D_HID), lambda r: (r, 0)),
            pl.BlockSpec((NW, _BR), lambda r: (0, r)),
            pl.BlockSpec((D_HID,), lambda r: (0,)),
            pl.BlockSpec((D_HID, D_OUT), lambda r: (0, 0)),
        ],
        out_specs=pl.BlockSpec((_BR, D_OUT), lambda r: (r, 0)),
    )(aggp, hp, degp, b1, W2)


def _t3_body(aggp_ref, hp_ref, degp_ref, b2_ref, out_ref):
    dinv = _dinv_from_parts(degp_ref[...])
    tot = aggp_ref[0] + aggp_ref[1] + hp_ref[...]
    o = tot * dinv[:, None] + b2_ref[...][None, :]
    m = jnp.max(o, axis=1, keepdims=True)
    lse = m + jnp.log(jnp.sum(jnp.exp(o - m), axis=1, keepdims=True))
    out_ref[...] = o - lse


def _t3(aggp, hp, degp, b2):
    return pl.pallas_call(
        _t3_body,
        out_shape=jax.ShapeDtypeStruct((N_NODES, D_OUT), jnp.float32),
        grid=(_NGRID,),
        in_specs=[
            pl.BlockSpec((NC, _BR, D_OUT), lambda r: (0, r, 0)),
            pl.BlockSpec((_BR, D_OUT), lambda r: (r, 0)),
            pl.BlockSpec((NW, _BR), lambda r: (0, r)),
            pl.BlockSpec((D_OUT,), lambda r: (0,)),
        ],
        out_specs=pl.BlockSpec((_BR, D_OUT), lambda r: (r, 0)),
    )(aggp, hp, degp, b2)


def kernel(x, edge_index, W1, b1, W2, b2):
    pad_e = EP - N_EDGES
    src_c = jnp.concatenate(
        [edge_index[0], jnp.full((pad_e,), N_NODES, jnp.int32)]
    ).reshape(TOTAL_CH, CHUNK)
    dst_flat = jnp.concatenate(
        [edge_index[1], jnp.full((pad_e,), N_NODES, jnp.int32)]
    )
    dst_c = dst_flat.reshape(TOTAL_CH, CHUNK)

    degp = _deg_kernel(dst_flat.reshape(NW, EPW0))
    h1p = _t1(x, W1, degp)
    agg1 = _agg128(h1p, src_c, dst_c)
    h2p = _t2(agg1, h1p, degp, b1, W2)
    agg2 = _agg16(h2p, src_c, dst_c)
    return _t3(agg2, h2p, degp, b2)


# speedup vs baseline: 1.0206x; 1.0070x over previous
import functools
import math

import jax
import jax.numpy as jnp
from jax import lax
from jax.experimental import pallas as pl
from jax.experimental.pallas import tpu as pltpu
from jax.experimental.pallas import tpu_sc as plsc

N_NODES = 10000
N_EDGES = 320000
D_IN = 128
D_HID = 128
D_OUT = 16

NC = 2
NS = 16
NW = NC * NS
NP = 10240
CHUNK = 120
CPT = 168
EP = NS * CPT * CHUNK
TOTAL_CH = EP // CHUNK
EPW0 = EP // NW
ROWS_PER_TILE = NP // NS
PART_BUF = 32
NBUF = 2

A0_128, A1_128 = 128, 40
A0_16, A1_16 = 96, 72

_mesh = plsc.VectorSubcoreMesh(core_axis_name="c", subcore_axis_name="s")
_sc_params = pltpu.CompilerParams(needs_layout_passes=False)


def _parts(n):
    k = math.ceil(n / PART_BUF)
    base = (n // k) & ~7
    sizes = [base] * k
    leftover = n - base * k
    i = 0
    while leftover > 0:
        sizes[i % k] += 8
        leftover -= 8
        i += 1
    assert sum(sizes) == n and all(p % 8 == 0 and p <= PART_BUF for p in sizes)
    return sizes


@functools.partial(
    pl.kernel,
    out_type=jax.ShapeDtypeStruct((NW, NP), jnp.float32),
    mesh=_mesh,
    scratch_types=[
        pltpu.VMEM((EPW0,), jnp.int32),
        pltpu.VMEM((NP,), jnp.float32),
        pltpu.SemaphoreType.DMA,
    ],
    compiler_params=_sc_params,
)
def _deg_kernel(dst_hbm, out_hbm, dst_v, deg_v, sem):
    c = lax.axis_index("c")
    s = lax.axis_index("s")
    wid = s * NC + c

    cp = pltpu.async_copy(dst_hbm.at[wid], dst_v, sem)

    zeros16 = jnp.zeros((16,), jnp.float32)

    def zero_body(i, _):
        deg_v[pl.ds(i * 16, 16)] = zeros16
        return None

    lax.fori_loop(0, NP // 16, zero_body, None)
    cp.wait()

    ones16 = jnp.ones((16,), jnp.float32)

    def add_body(i, _):
        idx = dst_v[pl.ds(i * 16, 16)]
        plsc.addupdate_scatter(deg_v, [idx], ones16)
        return None

    lax.fori_loop(0, EPW0 // 16, add_body, None)

    pltpu.sync_copy(deg_v, out_hbm.at[wid])


def _make_agg_kernel(D, a0, a1):
    params = (
        _sc_params
        if D % 128 == 0
        else pltpu.CompilerParams(
            needs_layout_passes=False, use_tc_tiling_on_sc=False
        )
    )
    assert NS * (a0 + a1) == TOTAL_CH

    @functools.partial(
        pl.kernel,
        out_type=jax.ShapeDtypeStruct((NC, NP, D), jnp.float32),
        mesh=_mesh,
        scratch_types=[
            pltpu.VMEM((2, PART_BUF, CHUNK), jnp.int32),
            pltpu.VMEM((2, PART_BUF, CHUNK), jnp.int32),
            pltpu.VMEM((CHUNK, D), jnp.float32),
            pltpu.VMEM((CHUNK, D), jnp.float32),
            pltpu.VMEM_SHARED((NP, D), jnp.float32),
            pltpu.SemaphoreType.DMA,
            pltpu.SemaphoreType.DMA,
            pltpu.SemaphoreType.DMA,
            pltpu.SemaphoreType.DMA,
            pltpu.SemaphoreType.DMA,
            pltpu.SemaphoreType.DMA,
        ],
        compiler_params=params,
    )
    def agg(hp_hbm, src_hbm, dst_hbm, out_hbm, src_v, dst_v, buf0, buf1,
            acc_s, gsem0, gsem1, ssem0, ssem1, sem_i0, sem_i1):
        bufs = (buf0, buf1)
        gsems = (gsem0, gsem1)
        ssems = (ssem0, ssem1)
        c = lax.axis_index("c")
        s = lax.axis_index("s")
        isem = (sem_i0, sem_i1)

        def idx_copies(slot, psz, chunk_start):
            return (
                pltpu.make_async_copy(
                    src_hbm.at[pl.ds(chunk_start, psz)],
                    src_v.at[slot, pl.ds(0, psz)],
                    isem[slot],
                ),
                pltpu.make_async_copy(
                    dst_hbm.at[pl.ds(chunk_start, psz)],
                    dst_v.at[slot, pl.ds(0, psz)],
                    isem[slot],
                ),
            )

        def fire_idx(slot, psz, chunk_start):
            for cp in idx_copies(slot, psz, chunk_start):
                cp.start()

        def wait_idx(slot, psz, chunk_start):
            for cp in idx_copies(slot, psz, chunk_start):
                cp.wait()

        parts0 = _parts(a0)
        parts1 = _parts(a1)
        base0 = s * a0
        base1 = NS * a0 + s * a1

        @pl.when(c == 0)
        def _():
            fire_idx(0, parts0[0], base0)

        @pl.when(c == 1)
        def _():
            fire_idx(0, parts1[0], base1)

        zeros16 = jnp.zeros((16,), jnp.float32)
        ncol = D // 16

        def zero_body(i, _):
            r = i // ncol
            k = i % ncol
            bufs[0][r, pl.ds(k * 16, 16)] = zeros16
            return None

        lax.fori_loop(0, CHUNK * ncol, zero_body, None)

        zsizes = []
        left = ROWS_PER_TILE
        while left > 0:
            zsizes.append(min(CHUNK, left))
            left -= zsizes[-1]
        off = 0
        for zs in zsizes:
            pltpu.sync_copy(
                bufs[0].at[pl.ds(0, zs)],
                acc_s.at[pl.ds(s * ROWS_PER_TILE + off, zs)],
            )
            off += zs
        plsc.subcore_barrier()

        def gather_cp(slot, j, b):
            return pltpu.make_async_copy(
                hp_hbm.at[src_v.at[slot, j]], bufs[b], gsems[b]
            )

        def scatter_cp(slot, j, b):
            return pltpu.make_async_copy(
                bufs[b], acc_s.at[dst_v.at[slot, j]], ssems[b]
            )

        def main_loop(parts, base):
            off = 0
            for pi, psz in enumerate(parts):
                slot = pi % 2
                wait_idx(slot, psz, base + off)
                if pi + 1 < len(parts):
                    fire_idx(1 - slot, parts[pi + 1], base + off + psz)

                for b in range(NBUF):
                    gather_cp(slot, b, b).start()

                def ring_body(i, _):
                    for b in range(NBUF):
                        j = NBUF * i + b
                        gather_cp(slot, j, b).wait()
                        cp = scatter_cp(slot, j, b)
                        cp.start(add=True)
                        cp.wait()

                        @pl.when(j + NBUF < psz)
                        def _():
                            gather_cp(slot, j + NBUF, b).start()

                    return None

                lax.fori_loop(0, psz // NBUF, ring_body, None)
                off += psz

        @pl.when(c == 0)
        def _():
            main_loop(parts0, base0)

        @pl.when(c == 1)
        def _():
            main_loop(parts1, base1)

        plsc.subcore_barrier()

        off = 0
        for zs in zsizes:
            base = s * ROWS_PER_TILE + off
            pltpu.sync_copy(acc_s.at[pl.ds(base, zs)], bufs[0].at[pl.ds(0, zs)])
            pltpu.sync_copy(bufs[0].at[pl.ds(0, zs)], out_hbm.at[c, pl.ds(base, zs)])
            off += zs

    return agg


_agg128 = _make_agg_kernel(D_HID, A0_128, A1_128)
_agg16 = _make_agg_kernel(D_OUT, A0_16, A1_16)


_BR = 1024
_NGRID = NP // _BR


def _dinv_from_parts(degp):
    deg = jnp.sum(degp, axis=0) + 1.0
    return lax.rsqrt(jnp.maximum(deg, 1.0))


def _t1_body(x_ref, w_ref, degp_ref, out_ref):
    dinv = _dinv_from_parts(degp_ref[...])
    h = jnp.dot(x_ref[...], w_ref[...], preferred_element_type=jnp.float32)
    out_ref[...] = h * dinv[:, None]


def _t1(x, W1, degp):
    return pl.pallas_call(
        _t1_body,
        out_shape=jax.ShapeDtypeStruct((NP, D_HID), jnp.float32),
        grid=(_NGRID,),
        in_specs=[
            pl.BlockSpec((_BR, D_IN), lambda r: (r, 0)),
            pl.BlockSpec((D_IN, D_HID), lambda r: (0, 0)),
            pl.BlockSpec((NW, _BR), lambda r: (0, r)),
        ],
        out_specs=pl.BlockSpec((_BR, D_HID), lambda r: (r, 0)),
    )(x, W1, degp)


def _t2_body(aggp_ref, hp_ref, degp_ref, b1_ref, w2_ref, out_ref):
    dinv = _dinv_from_parts(degp_ref[...])
    tot = aggp_ref[0] + aggp_ref[1] + hp_ref[...]
    a = tot * dinv[:, None] + b1_ref[...][None, :]
    h1 = jnp.maximum(a, 0.0)
    h2 = jnp.dot(h1, w2_ref[...], preferred_element_type=jnp.float32)
    out_ref[...] = h2 * dinv[:, None]


def _t2(aggp, hp, degp, b1, W2):
    return pl.pallas_call(
        _t2_body,
        out_shape=jax.ShapeDtypeStruct((NP, D_OUT), jnp.float32),
        grid=(_NGRID,),
        in_specs=[
            pl.BlockSpec((NC, _BR, D_HID), lambda r: (0, r, 0)),
            pl.BlockSpec((_BR, D_HID), lambda r: (r, 0)),
            pl.BlockSpec((NW, _BR), lambda r: (0, r)),
            pl.BlockSpec((D_HID,), lambda r: (0,)),
            pl.BlockSpec((D_HID, D_OUT), lambda r: (0, 0)),
        ],
        out_specs=pl.BlockSpec((_BR, D_OUT), lambda r: (r, 0)),
    )(aggp, hp, degp, b1, W2)


def _t3_body(aggp_ref, hp_ref, degp_ref, b2_ref, out_ref):
    dinv = _dinv_from_parts(degp_ref[...])
    tot = aggp_ref[0] + aggp_ref[1] + hp_ref[...]
    o = tot * dinv[:, None] + b2_ref[...][None, :]
    m = jnp.max(o, axis=1, keepdims=True)
    lse = m + jnp.log(jnp.sum(jnp.exp(o - m), axis=1, keepdims=True))
    out_ref[...] = o - lse


def _t3(aggp, hp, degp, b2):
    return pl.pallas_call(
        _t3_body,
        out_shape=jax.ShapeDtypeStruct((N_NODES, D_OUT), jnp.float32),
        grid=(_NGRID,),
        in_specs=[
            pl.BlockSpec((NC, _BR, D_OUT), lambda r: (0, r, 0)),
            pl.BlockSpec((_BR, D_OUT), lambda r: (r, 0)),
            pl.BlockSpec((NW, _BR), lambda r: (0, r)),
            pl.BlockSpec((D_OUT,), lambda r: (0,)),
        ],
        out_specs=pl.BlockSpec((_BR, D_OUT), lambda r: (r, 0)),
    )(aggp, hp, degp, b2)


def kernel(x, edge_index, W1, b1, W2, b2):
    pad_e = EP - N_EDGES
    src_c = jnp.concatenate(
        [edge_index[0], jnp.full((pad_e,), N_NODES, jnp.int32)]
    ).reshape(TOTAL_CH, CHUNK)
    dst_flat = jnp.concatenate(
        [edge_index[1], jnp.full((pad_e,), N_NODES, jnp.int32)]
    )
    dst_c = dst_flat.reshape(TOTAL_CH, CHUNK)

    degp = _deg_kernel(dst_flat.reshape(NW, EPW0))
    h1p = _t1(x, W1, degp)
    agg1 = _agg128(h1p, src_c, dst_c)
    h2p = _t2(agg1, h1p, degp, b1, W2)
    agg2 = _agg16(h2p, src_c, dst_c)
    return _t3(agg2, h2p, degp, b2)
